# Initial kernel scaffold; baseline (speedup 1.0000x reference)
import jax
import jax.numpy as jnp
from jax.experimental import pallas as pl


def kernel(output, labels):
    raise NotImplementedError("write your pallas kernel here")


# speedup vs baseline: 1.6373x; 1.6373x over previous
import jax
import jax.numpy as jnp
from jax import lax
from jax.experimental import pallas as pl
from jax.experimental.pallas import tpu as pltpu

_ROWS = 5184
_COLS = 640
_BLK = 64
_GRID = _ROWS // _BLK
_GPR = _COLS // 5
_KMAX = 32


def _fold(x):
    x = (x[:, 0:128] + x[:, 128:256] + x[:, 256:384]
         + x[:, 384:512] + x[:, 512:640])
    return x.reshape(_BLK // 8, 8, _GPR).sum(axis=0)


def _body(o_ref, l_ref, out_ref, sel_ref, scores_ref, segmax_ref, acc_ref):
    pid = pl.program_id(0)

    @pl.when(pid == 0)
    def _init():
        acc_ref[...] = jnp.zeros_like(acc_ref)
        ri = lax.broadcasted_iota(jnp.int32, (_COLS, _GPR), 0)
        ci = lax.broadcasted_iota(jnp.int32, (_COLS, _GPR), 1)
        sel_ref[...] = (ri == 5 * ci).astype(jnp.float32)

    o = o_ref[...]
    l = l_ref[...]
    lane = lax.broadcasted_iota(jnp.int32, (_BLK, _COLS), 1)
    ch = lax.rem(lane, 5)
    is_head = ch == 0
    posh = jnp.logical_and(is_head, l > 0.5)
    negh = jnp.logical_and(is_head, l < -0.5)
    poshf = posh.astype(jnp.float32)
    neghf = negh.astype(jnp.float32)

    spread = poshf
    sh = poshf
    for s in range(1, 5):
        sh = jnp.roll(sh, 1, axis=1)
        sh = jnp.where(lane >= 1, sh, 0.0)
        spread = spread + sh
    regmask = jnp.where(is_head, 0.0, spread)

    mlogp = jnp.where(posh, -jnp.log(o), 0.0)
    d = o - l
    ad = jnp.abs(d)
    sl1 = jnp.where(ad < 1.0, 0.5 * d * d, ad - 0.5)

    acc_ref[0:8, :] = acc_ref[0:8, :] + _fold(mlogp)
    acc_ref[8:16, :] = acc_ref[8:16, :] + _fold(sl1 * regmask)
    acc_ref[16:24, :] = acc_ref[16:24, :] + _fold(poshf)
    acc_ref[24:32, :] = acc_ref[24:32, :] + _fold(neghf)

    negs = jnp.where(negh, o, 0.0)
    compact = lax.dot_general(
        negs, sel_ref[...], (((1,), (0,)), ((), ())),
        precision=lax.Precision.HIGHEST,
        preferred_element_type=jnp.float32)
    compact = jnp.where(compact > 1e-4, compact, -1.0)
    scores_ref[pl.ds(pid * _BLK, _BLK), :] = compact
    segmax_ref[pl.ds(pid, 1), :] = jnp.full((1, _GPR), jnp.max(compact))

    @pl.when(pid == _GRID - 1)
    def _fin():
        logp_sum = jnp.sum(acc_ref[0:8, :])
        sl1_sum = jnp.sum(acc_ref[8:16, :])
        posc = jnp.sum(acc_ref[16:24, :])
        negc = jnp.sum(acc_ref[24:32, :])
        kf = jnp.minimum(jnp.float32(_KMAX), negc)

        rowid = lax.broadcasted_iota(jnp.int32, (_GRID, _GPR), 0)

        def step(_, carry):
            rem, acc = carry
            sm = segmax_ref[...]
            m = jnp.max(sm)
            s = jnp.min(jnp.where(sm == m, rowid, _GRID))
            seg = scores_ref[pl.ds(s * _BLK, _BLK), :]
            eq = seg == m
            cnt = jnp.sum(eq.astype(jnp.float32))
            valid = m > -0.5
            take = jnp.where(valid, jnp.minimum(cnt, rem), 0.0)
            acc = acc + take * (-jnp.log(1.0 - m))
            rem = rem - take
            newseg = jnp.where(eq, -1.0, seg)
            scores_ref[pl.ds(s * _BLK, _BLK), :] = newseg
            segmax_ref[pl.ds(s, 1), :] = jnp.full((1, _GPR), jnp.max(newseg))
            return rem, acc

        _, negsum = lax.fori_loop(0, _KMAX, step, (kf, jnp.float32(0.0)))
        loss = 0.5 * logp_sum / posc + 0.5 * negsum / kf + sl1_sum / posc
        out_ref[...] = jnp.full((1, 1), loss)


@jax.jit
def kernel(output, labels):
    o2 = output.reshape(_ROWS, _COLS)
    l2 = labels.reshape(_ROWS, _COLS)
    out = pl.pallas_call(
        _body,
        grid=(_GRID,),
        in_specs=[pl.BlockSpec((_BLK, _COLS), lambda i: (i, 0)),
                  pl.BlockSpec((_BLK, _COLS), lambda i: (i, 0))],
        out_specs=pl.BlockSpec((1, 1), lambda i: (0, 0)),
        out_shape=jax.ShapeDtypeStruct((1, 1), jnp.float32),
        scratch_shapes=[
            pltpu.VMEM((_COLS, _GPR), jnp.float32),
            pltpu.VMEM((_ROWS, _GPR), jnp.float32),
            pltpu.VMEM((_GRID, _GPR), jnp.float32),
            pltpu.VMEM((32, _GPR), jnp.float32),
        ],
    )(o2, l2)
    return out[0, 0]


# speedup vs baseline: 10.1451x; 6.1964x over previous
import jax
import jax.numpy as jnp
from jax import lax
from jax.experimental import pallas as pl
from jax.experimental.pallas import tpu as pltpu

_ROWS = 5184
_BLK = 64
_GRID = _ROWS // _BLK
_KMAX = 32


def _fold(x):
    return x.reshape(_BLK // 8, 8, 128).sum(axis=0)


def _body(o_ref, l_ref, out_ref, scores_ref, segmax_ref, acc_ref):
    pid = pl.program_id(0)

    @pl.when(pid == 0)
    def _init():
        acc_ref[...] = jnp.zeros_like(acc_ref)

    o0 = o_ref[0]
    l0 = l_ref[0]
    posm = l0 > 0.5
    posf = posm.astype(jnp.float32)
    negm = l0 < -0.5

    mlogp = jnp.where(posm, -jnp.log(o0), 0.0)

    sl1 = jnp.zeros_like(o0)
    for c in range(1, 5):
        d = o_ref[c] - l_ref[c]
        ad = jnp.abs(d)
        sl1 = sl1 + jnp.where(ad < 1.0, 0.5 * d * d, ad - 0.5)

    acc_ref[0:8, :] = acc_ref[0:8, :] + _fold(mlogp)
    acc_ref[8:16, :] = acc_ref[8:16, :] + _fold(sl1 * posf)
    acc_ref[16:24, :] = acc_ref[16:24, :] + _fold(posf)
    acc_ref[24:32, :] = acc_ref[24:32, :] + _fold(negm.astype(jnp.float32))

    scores = jnp.where(negm, o0, -1.0)
    scores_ref[pl.ds(pid * _BLK, _BLK), :] = scores
    segmax_ref[pl.ds(pid, 1), :] = jnp.full((1, 128), jnp.max(scores))

    @pl.when(pid == _GRID - 1)
    def _fin():
        logp_sum = jnp.sum(acc_ref[0:8, :])
        sl1_sum = jnp.sum(acc_ref[8:16, :])
        posc = jnp.sum(acc_ref[16:24, :])
        negc = jnp.sum(acc_ref[24:32, :])
        kf = jnp.minimum(jnp.float32(_KMAX), negc)

        rowid = lax.broadcasted_iota(jnp.int32, (_GRID, 128), 0)

        def step(_, carry):
            rem, acc = carry
            sm = segmax_ref[...]
            m = jnp.max(sm)
            s = jnp.min(jnp.where(sm == m, rowid, _GRID))
            seg = scores_ref[pl.ds(s * _BLK, _BLK), :]
            eq = seg == m
            cnt = jnp.sum(eq.astype(jnp.float32))
            valid = m > -0.5
            take = jnp.where(valid, jnp.minimum(cnt, rem), 0.0)
            acc = acc + take * (-jnp.log(1.0 - m))
            rem = rem - take
            newseg = jnp.where(eq, -1.0, seg)
            scores_ref[pl.ds(s * _BLK, _BLK), :] = newseg
            segmax_ref[pl.ds(s, 1), :] = jnp.full((1, 128), jnp.max(newseg))
            return rem, acc

        _, negsum = lax.fori_loop(0, _KMAX, step, (kf, jnp.float32(0.0)))
        loss = 0.5 * logp_sum / posc + 0.5 * negsum / kf + sl1_sum / posc
        out_ref[...] = jnp.full((1, 1), loss)


@jax.jit
def kernel(output, labels):
    ot = jnp.moveaxis(output, 2, 0).reshape(5, _ROWS, 128)
    lt = jnp.moveaxis(labels, 2, 0).reshape(5, _ROWS, 128)
    out = pl.pallas_call(
        _body,
        grid=(_GRID,),
        in_specs=[pl.BlockSpec((5, _BLK, 128), lambda i: (0, i, 0)),
                  pl.BlockSpec((5, _BLK, 128), lambda i: (0, i, 0))],
        out_specs=pl.BlockSpec((1, 1), lambda i: (0, 0)),
        out_shape=jax.ShapeDtypeStruct((1, 1), jnp.float32),
        scratch_shapes=[
            pltpu.VMEM((_ROWS, 128), jnp.float32),
            pltpu.VMEM((_GRID, 128), jnp.float32),
            pltpu.VMEM((32, 128), jnp.float32),
        ],
    )(ot, lt)
    return out[0, 0]


# speedup vs baseline: 10.7528x; 1.0599x over previous
import jax
import jax.numpy as jnp
from jax import lax
from jax.experimental import pallas as pl
from jax.experimental.pallas import tpu as pltpu

_ROWS = 5184
_BLK = 64
_GRID = _ROWS // _BLK
_KMAX = 32


def _fold(x):
    return x.reshape(_BLK // 8, 8, 128).sum(axis=0)


def _body(o_ref, l_ref, out_ref, scores_ref, segmax_ref, acc_ref):
    pid = pl.program_id(0)

    @pl.when(pid == 0)
    def _init():
        acc_ref[...] = jnp.zeros_like(acc_ref)

    o0 = o_ref[0]
    l0 = l_ref[0]
    posm = l0 > 0.5
    posf = posm.astype(jnp.float32)
    negm = l0 < -0.5

    mlogp = jnp.where(posm, -jnp.log(o0), 0.0)

    sl1 = jnp.zeros_like(o0)
    for c in range(1, 5):
        d = o_ref[c] - l_ref[c]
        ad = jnp.abs(d)
        sl1 = sl1 + jnp.where(ad < 1.0, 0.5 * d * d, ad - 0.5)

    acc_ref[0:8, :] = acc_ref[0:8, :] + _fold(mlogp)
    acc_ref[8:16, :] = acc_ref[8:16, :] + _fold(sl1 * posf)
    acc_ref[16:24, :] = acc_ref[16:24, :] + _fold(posf)
    acc_ref[24:32, :] = acc_ref[24:32, :] + _fold(negm.astype(jnp.float32))

    scores = jnp.where(negm, o0, -1.0)
    scores_ref[pl.ds(pid * _BLK, _BLK), :] = scores
    segmax_ref[pl.ds(pid, 1), :] = jnp.max(scores, axis=0, keepdims=True)

    @pl.when(pid == _GRID - 1)
    def _fin():
        logp_sum = jnp.sum(acc_ref[0:8, :])
        sl1_sum = jnp.sum(acc_ref[8:16, :])
        posc = jnp.sum(acc_ref[16:24, :])
        negc = jnp.sum(acc_ref[24:32, :])
        kf = jnp.minimum(jnp.float32(_KMAX), negc)

        rowid = lax.broadcasted_iota(jnp.int32, (_GRID, 128), 0)

        def step(_, carry):
            rem, acc = carry
            sm = segmax_ref[...]
            m = jnp.max(sm)
            s = jnp.min(jnp.where(sm == m, rowid, _GRID))
            seg = scores_ref[pl.ds(s * _BLK, _BLK), :]
            eq = seg == m
            cnt = jnp.sum(eq.astype(jnp.float32))
            valid = m > -0.5
            take = jnp.where(valid, jnp.minimum(cnt, rem), 0.0)
            acc = acc + take * (-jnp.log(1.0 - m))
            rem = rem - take
            newseg = jnp.where(eq, -1.0, seg)
            scores_ref[pl.ds(s * _BLK, _BLK), :] = newseg
            segmax_ref[pl.ds(s, 1), :] = jnp.max(newseg, axis=0, keepdims=True)
            return rem, acc

        _, negsum = lax.fori_loop(0, _KMAX, step, (kf, jnp.float32(0.0)))
        loss = 0.5 * logp_sum / posc + 0.5 * negsum / kf + sl1_sum / posc
        out_ref[...] = jnp.full((1, 1), loss)


@jax.jit
def kernel(output, labels):
    ot = jnp.moveaxis(output, 2, 0).reshape(5, _ROWS, 128)
    lt = jnp.moveaxis(labels, 2, 0).reshape(5, _ROWS, 128)
    out = pl.pallas_call(
        _body,
        grid=(_GRID,),
        in_specs=[pl.BlockSpec((5, _BLK, 128), lambda i: (0, i, 0)),
                  pl.BlockSpec((5, _BLK, 128), lambda i: (0, i, 0))],
        out_specs=pl.BlockSpec((1, 1), lambda i: (0, 0)),
        out_shape=jax.ShapeDtypeStruct((1, 1), jnp.float32),
        scratch_shapes=[
            pltpu.VMEM((_ROWS, 128), jnp.float32),
            pltpu.VMEM((_GRID, 128), jnp.float32),
            pltpu.VMEM((32, 128), jnp.float32),
        ],
    )(ot, lt)
    return out[0, 0]


# speedup vs baseline: 14.8453x; 1.3806x over previous
import jax
import jax.numpy as jnp
from jax import lax
from jax.experimental import pallas as pl
from jax.experimental.pallas import tpu as pltpu

_ROWS = 5184
_BLK = 192
_GRID = _ROWS // _BLK
_SEG = 64
_SPB = _BLK // _SEG
_NSEG = _ROWS // _SEG
_KMAX = 32


def _fold(x):
    return x.reshape(_BLK // 8, 8, 128).sum(axis=0)


def _body(o_ref, l_ref, out_ref, scores_ref, segmax_ref, acc_ref):
    pid = pl.program_id(0)

    @pl.when(pid == 0)
    def _init():
        acc_ref[...] = jnp.zeros_like(acc_ref)

    o0 = o_ref[0]
    l0 = l_ref[0]
    posm = l0 > 0.5
    posf = posm.astype(jnp.float32)
    negm = l0 < -0.5

    mlogp = jnp.where(posm, -jnp.log(o0), 0.0)

    sl1 = jnp.zeros_like(o0)
    for c in range(1, 5):
        d = o_ref[c] - l_ref[c]
        ad = jnp.abs(d)
        sl1 = sl1 + jnp.where(ad < 1.0, 0.5 * d * d, ad - 0.5)

    acc_ref[0:8, :] = acc_ref[0:8, :] + _fold(mlogp)
    acc_ref[8:16, :] = acc_ref[8:16, :] + _fold(sl1 * posf)
    acc_ref[16:24, :] = acc_ref[16:24, :] + _fold(posf)
    acc_ref[24:32, :] = acc_ref[24:32, :] + _fold(negm.astype(jnp.float32))

    scores = jnp.where(negm, o0, -1.0)
    scores_ref[pl.ds(pid * _BLK, _BLK), :] = scores
    for j in range(_SPB):
        segmax_ref[pl.ds(pid * _SPB + j, 1), :] = jnp.max(
            scores[j * _SEG:(j + 1) * _SEG], axis=0, keepdims=True)

    @pl.when(pid == _GRID - 1)
    def _fin():
        logp_sum = jnp.sum(acc_ref[0:8, :])
        sl1_sum = jnp.sum(acc_ref[8:16, :])
        posc = jnp.sum(acc_ref[16:24, :])
        negc = jnp.sum(acc_ref[24:32, :])
        kf = jnp.minimum(jnp.float32(_KMAX), negc)

        rowid = lax.broadcasted_iota(jnp.int32, (_NSEG, 128), 0)

        def step(_, carry):
            rem, acc = carry
            sm = segmax_ref[...]
            m = jnp.max(sm)
            s = jnp.min(jnp.where(sm == m, rowid, _NSEG))
            seg = scores_ref[pl.ds(s * _SEG, _SEG), :]
            eq = seg == m
            cnt = jnp.sum(eq.astype(jnp.float32))
            valid = m > -0.5
            take = jnp.where(valid, jnp.minimum(cnt, rem), 0.0)
            acc = acc + take * (-jnp.log(1.0 - m))
            rem = rem - take
            newseg = jnp.where(eq, -1.0, seg)
            scores_ref[pl.ds(s * _SEG, _SEG), :] = newseg
            segmax_ref[pl.ds(s, 1), :] = jnp.max(newseg, axis=0, keepdims=True)
            return rem, acc

        _, negsum = lax.fori_loop(0, _KMAX, step, (kf, jnp.float32(0.0)))
        loss = 0.5 * logp_sum / posc + 0.5 * negsum / kf + sl1_sum / posc
        out_ref[...] = jnp.full((1, 1), loss)


@jax.jit
def kernel(output, labels):
    ot = jnp.moveaxis(output, 2, 0).reshape(5, _ROWS, 128)
    lt = jnp.moveaxis(labels, 2, 0).reshape(5, _ROWS, 128)
    out = pl.pallas_call(
        _body,
        grid=(_GRID,),
        in_specs=[pl.BlockSpec((5, _BLK, 128), lambda i: (0, i, 0)),
                  pl.BlockSpec((5, _BLK, 128), lambda i: (0, i, 0))],
        out_specs=pl.BlockSpec((1, 1), lambda i: (0, 0)),
        out_shape=jax.ShapeDtypeStruct((1, 1), jnp.float32),
        scratch_shapes=[
            pltpu.VMEM((_ROWS, 128), jnp.float32),
            pltpu.VMEM((_NSEG, 128), jnp.float32),
            pltpu.VMEM((32, 128), jnp.float32),
        ],
    )(ot, lt)
    return out[0, 0]


# speedup vs baseline: 16.9780x; 1.1437x over previous
import jax
import jax.numpy as jnp
from jax import lax
from jax.experimental import pallas as pl
from jax.experimental.pallas import tpu as pltpu

_ROWS = 5184
_BLK = 576
_GRID = _ROWS // _BLK
_SEG = 64
_SPB = _BLK // _SEG
_NSEG = _ROWS // _SEG
_KMAX = 32


def _fold(x):
    return x.reshape(_BLK // 8, 8, 128).sum(axis=0)


def _body(o_ref, l_ref, out_ref, scores_ref, segmax_ref, acc_ref):
    pid = pl.program_id(0)

    @pl.when(pid == 0)
    def _init():
        acc_ref[...] = jnp.zeros_like(acc_ref)

    o0 = o_ref[0]
    l0 = l_ref[0]
    posm = l0 > 0.5
    posf = posm.astype(jnp.float32)
    negm = l0 < -0.5

    mlogp = jnp.where(posm, -jnp.log(o0), 0.0)

    sl1 = jnp.zeros_like(o0)
    for c in range(1, 5):
        d = o_ref[c] - l_ref[c]
        ad = jnp.abs(d)
        sl1 = sl1 + jnp.where(ad < 1.0, 0.5 * d * d, ad - 0.5)

    acc_ref[0:8, :] = acc_ref[0:8, :] + _fold(mlogp)
    acc_ref[8:16, :] = acc_ref[8:16, :] + _fold(sl1 * posf)
    acc_ref[16:24, :] = acc_ref[16:24, :] + _fold(posf)
    acc_ref[24:32, :] = acc_ref[24:32, :] + _fold(negm.astype(jnp.float32))

    scores = jnp.where(negm, o0, -1.0)
    scores_ref[pl.ds(pid * _BLK, _BLK), :] = scores
    for j in range(_SPB):
        segmax_ref[pl.ds(pid * _SPB + j, 1), :] = jnp.max(
            scores[j * _SEG:(j + 1) * _SEG], axis=0, keepdims=True)

    @pl.when(pid == _GRID - 1)
    def _fin():
        logp_sum = jnp.sum(acc_ref[0:8, :])
        sl1_sum = jnp.sum(acc_ref[8:16, :])
        posc = jnp.sum(acc_ref[16:24, :])
        negc = jnp.sum(acc_ref[24:32, :])
        kf = jnp.minimum(jnp.float32(_KMAX), negc)

        rowid = lax.broadcasted_iota(jnp.int32, (_NSEG, 128), 0)

        def step(_, carry):
            rem, acc = carry
            sm = segmax_ref[...]
            m = jnp.max(sm)
            s = jnp.min(jnp.where(sm == m, rowid, _NSEG))
            seg = scores_ref[pl.ds(s * _SEG, _SEG), :]
            eq = seg == m
            cnt = jnp.sum(eq.astype(jnp.float32))
            valid = m > -0.5
            take = jnp.where(valid, jnp.minimum(cnt, rem), 0.0)
            acc = acc + take * (-jnp.log(1.0 - m))
            rem = rem - take
            newseg = jnp.where(eq, -1.0, seg)
            scores_ref[pl.ds(s * _SEG, _SEG), :] = newseg
            segmax_ref[pl.ds(s, 1), :] = jnp.max(newseg, axis=0, keepdims=True)
            return rem, acc

        _, negsum = lax.fori_loop(0, _KMAX, step, (kf, jnp.float32(0.0)))
        loss = 0.5 * logp_sum / posc + 0.5 * negsum / kf + sl1_sum / posc
        out_ref[...] = jnp.full((1, 1), loss)


@jax.jit
def kernel(output, labels):
    ot = jnp.moveaxis(output, 2, 0).reshape(5, _ROWS, 128)
    lt = jnp.moveaxis(labels, 2, 0).reshape(5, _ROWS, 128)
    out = pl.pallas_call(
        _body,
        grid=(_GRID,),
        in_specs=[pl.BlockSpec((5, _BLK, 128), lambda i: (0, i, 0)),
                  pl.BlockSpec((5, _BLK, 128), lambda i: (0, i, 0))],
        out_specs=pl.BlockSpec((1, 1), lambda i: (0, 0)),
        out_shape=jax.ShapeDtypeStruct((1, 1), jnp.float32),
        scratch_shapes=[
            pltpu.VMEM((_ROWS, 128), jnp.float32),
            pltpu.VMEM((_NSEG, 128), jnp.float32),
            pltpu.VMEM((32, 128), jnp.float32),
        ],
    )(ot, lt)
    return out[0, 0]


# speedup vs baseline: 17.2845x; 1.0181x over previous
import jax
import jax.numpy as jnp
from jax import lax
from jax.experimental import pallas as pl
from jax.experimental.pallas import tpu as pltpu

_ROWS = 5184
_BLK = 1728
_GRID = _ROWS // _BLK
_SEG = 64
_SPB = _BLK // _SEG
_NSEG = _ROWS // _SEG
_KMAX = 32


def _fold(x):
    return x.reshape(_BLK // 8, 8, 128).sum(axis=0)


def _body(o_ref, l_ref, out_ref, scores_ref, segmax_ref, acc_ref):
    pid = pl.program_id(0)

    @pl.when(pid == 0)
    def _init():
        acc_ref[...] = jnp.zeros_like(acc_ref)

    o0 = o_ref[0]
    l0 = l_ref[0]
    posm = l0 > 0.5
    posf = posm.astype(jnp.float32)
    negm = l0 < -0.5

    mlogp = jnp.where(posm, -jnp.log(o0), 0.0)

    sl1 = jnp.zeros_like(o0)
    for c in range(1, 5):
        d = o_ref[c] - l_ref[c]
        ad = jnp.abs(d)
        sl1 = sl1 + jnp.where(ad < 1.0, 0.5 * d * d, ad - 0.5)

    acc_ref[0:8, :] = acc_ref[0:8, :] + _fold(mlogp)
    acc_ref[8:16, :] = acc_ref[8:16, :] + _fold(sl1 * posf)
    acc_ref[16:24, :] = acc_ref[16:24, :] + _fold(posf)
    acc_ref[24:32, :] = acc_ref[24:32, :] + _fold(negm.astype(jnp.float32))

    scores = jnp.where(negm, o0, -1.0)
    scores_ref[pl.ds(pid * _BLK, _BLK), :] = scores
    for j in range(_SPB):
        segmax_ref[pl.ds(pid * _SPB + j, 1), :] = jnp.max(
            scores[j * _SEG:(j + 1) * _SEG], axis=0, keepdims=True)

    @pl.when(pid == _GRID - 1)
    def _fin():
        logp_sum = jnp.sum(acc_ref[0:8, :])
        sl1_sum = jnp.sum(acc_ref[8:16, :])
        posc = jnp.sum(acc_ref[16:24, :])
        negc = jnp.sum(acc_ref[24:32, :])
        kf = jnp.minimum(jnp.float32(_KMAX), negc)

        rowid = lax.broadcasted_iota(jnp.int32, (_NSEG, 128), 0)

        def step(_, carry):
            rem, acc = carry
            sm = segmax_ref[...]
            m = jnp.max(sm)
            s = jnp.min(jnp.where(sm == m, rowid, _NSEG))
            seg = scores_ref[pl.ds(s * _SEG, _SEG), :]
            eq = seg == m
            cnt = jnp.sum(eq.astype(jnp.float32))
            valid = m > -0.5
            take = jnp.where(valid, jnp.minimum(cnt, rem), 0.0)
            acc = acc + take * (-jnp.log(1.0 - m))
            rem = rem - take
            newseg = jnp.where(eq, -1.0, seg)
            scores_ref[pl.ds(s * _SEG, _SEG), :] = newseg
            segmax_ref[pl.ds(s, 1), :] = jnp.max(newseg, axis=0, keepdims=True)
            return rem, acc

        _, negsum = lax.fori_loop(0, _KMAX, step, (kf, jnp.float32(0.0)))
        loss = 0.5 * logp_sum / posc + 0.5 * negsum / kf + sl1_sum / posc
        out_ref[...] = jnp.full((1, 1), loss)


@jax.jit
def kernel(output, labels):
    ot = jnp.moveaxis(output, 2, 0).reshape(5, _ROWS, 128)
    lt = jnp.moveaxis(labels, 2, 0).reshape(5, _ROWS, 128)
    out = pl.pallas_call(
        _body,
        grid=(_GRID,),
        in_specs=[pl.BlockSpec((5, _BLK, 128), lambda i: (0, i, 0)),
                  pl.BlockSpec((5, _BLK, 128), lambda i: (0, i, 0))],
        out_specs=pl.BlockSpec((1, 1), lambda i: (0, 0)),
        out_shape=jax.ShapeDtypeStruct((1, 1), jnp.float32),
        scratch_shapes=[
            pltpu.VMEM((_ROWS, 128), jnp.float32),
            pltpu.VMEM((_NSEG, 128), jnp.float32),
            pltpu.VMEM((32, 128), jnp.float32),
        ],
    )(ot, lt)
    return out[0, 0]
